# Initial kernel scaffold; baseline (speedup 1.0000x reference)
import jax
import jax.numpy as jnp
from jax.experimental import pallas as pl


def kernel(x, edge_index, edge_weight, batch, W1, b1, W2, b2):
    raise NotImplementedError("write your pallas kernel here")


# speedup vs baseline: 10.2553x; 10.2553x over previous
import jax
import jax.numpy as jnp
from jax import lax
from jax.experimental import pallas as pl
from jax.experimental.pallas import tpu as pltpu
from jax.experimental.pallas import tpu_sc as plsc

N_NODES = 10000
N_EDGES = 320000
D = 128
G = 64

NC = 2
NS = 16
NW = NC * NS
K = 128
CP = 2560
EP = CP * K
PW = CP // NW
SB = 16
NOUT = PW // SB
NP = 10240
TROWS = NP // NS
BR = 2048
NB = NP // BR

_mesh = plsc.VectorSubcoreMesh(core_axis_name="c", subcore_axis_name="s")


def _deg_body(col_hbm, ew_hbm, z_hbm, out_hbm, colv, ewv, dacc):
    c = lax.axis_index("c")
    s = lax.axis_index("s")
    wid = s * NC + c
    pltpu.sync_copy(z_hbm.at[pl.ds(s * TROWS, TROWS)],
                    dacc.at[pl.ds(s * TROWS, TROWS)])
    plsc.subcore_barrier()

    def outer(i, carry):
        cb = wid * PW + i * SB
        pltpu.sync_copy(col_hbm.at[pl.ds(cb, SB)], colv)
        pltpu.sync_copy(ew_hbm.at[pl.ds(cb, SB)], ewv)

        def inner(j, carry2):
            pltpu.sync_copy(ewv.at[j], dacc.at[colv.at[j]], add=True)
            return carry2

        return lax.fori_loop(0, SB, inner, carry)

    lax.fori_loop(0, NOUT, outer, 0)
    plsc.subcore_barrier()
    pltpu.sync_copy(dacc.at[pl.ds(s * TROWS, TROWS)],
                    out_hbm.at[c, pl.ds(s * TROWS, TROWS)])


_deg_kernel = pl.kernel(
    _deg_body,
    out_type=jax.ShapeDtypeStruct((NC, NP), jnp.float32),
    mesh=_mesh,
    scratch_types=[
        pltpu.VMEM((SB, K), jnp.int32),
        pltpu.VMEM((SB, K), jnp.float32),
        pltpu.VMEM_SHARED((NP,), jnp.float32),
    ],
)


def _make_msg_body():
    def body(row_hbm, col_hbm, ew_hbm, hp_hbm, z_hbm, out_hbm,
             rowv, colv, ewv, rows0, rows1,
             gsem0, gsem1, ssem0, ssem1, acc):
        c = lax.axis_index("c")
        s = lax.axis_index("s")
        wid = s * NC + c
        pltpu.sync_copy(z_hbm.at[pl.ds(s * TROWS, TROWS)],
                        acc.at[pl.ds(s * TROWS, TROWS)])
        plsc.subcore_barrier()
        bufs = (rows0, rows1)
        gsems = (gsem0, gsem1)
        ssems = (ssem0, ssem1)

        def scale(buf, j):
            def grp(eb, carry):
                wv = ewv[j, pl.ds(eb * 16, 16)]
                for l in range(16):
                    w = wv[l]
                    e = eb * 16 + l
                    for q in range(D // 16):
                        buf[e, pl.ds(q * 16, 16)] = (
                            buf[e, pl.ds(q * 16, 16)] * w)
                return carry
            lax.fori_loop(0, K // 16, grp, 0)

        def outer(i, carry):
            cb = wid * PW + i * SB
            pltpu.sync_copy(row_hbm.at[pl.ds(cb, SB)], rowv)
            pltpu.sync_copy(col_hbm.at[pl.ds(cb, SB)], colv)
            pltpu.sync_copy(ew_hbm.at[pl.ds(cb, SB)], ewv)

            pg = [None, None]
            ps = [None, None]
            pg[0] = pltpu.async_copy(hp_hbm.at[rowv.at[0]], bufs[0], gsems[0])
            for j in range(SB):
                b = j % 2
                if j + 1 < SB:
                    nb = (j + 1) % 2
                    if ps[nb] is not None:
                        ps[nb].wait()
                        ps[nb] = None
                    pg[nb] = pltpu.async_copy(
                        hp_hbm.at[rowv.at[j + 1]], bufs[nb], gsems[nb])
                pg[b].wait()
                scale(bufs[b], j)
                ps[b] = pltpu.async_copy(
                    bufs[b], acc.at[colv.at[j]], ssems[b], add=True)
            for b in range(2):
                if ps[b] is not None:
                    ps[b].wait()
            return carry

        lax.fori_loop(0, NOUT, outer, 0)
        plsc.subcore_barrier()
        pltpu.sync_copy(acc.at[pl.ds(s * TROWS, TROWS)],
                        out_hbm.at[c, pl.ds(s * TROWS, TROWS)])
    return body


_msg_kernel = pl.kernel(
    _make_msg_body(),
    out_type=jax.ShapeDtypeStruct((NC, NP, D), jnp.float32),
    mesh=_mesh,
    scratch_types=[
        pltpu.VMEM((SB, K), jnp.int32),
        pltpu.VMEM((SB, K), jnp.int32),
        pltpu.VMEM((SB, K), jnp.float32),
        pltpu.VMEM((K, D), jnp.float32),
        pltpu.VMEM((K, D), jnp.float32),
        pltpu.SemaphoreType.DMA,
        pltpu.SemaphoreType.DMA,
        pltpu.SemaphoreType.DMA,
        pltpu.SemaphoreType.DMA,
        pltpu.VMEM_SHARED((NP, D), jnp.float32),
    ],
)


def _mm_body(x_ref, w_ref, o_ref):
    o_ref[...] = jnp.dot(x_ref[...], w_ref[...],
                         preferred_element_type=jnp.float32)


def _mm(x, w):
    return pl.pallas_call(
        _mm_body,
        grid=(NB,),
        in_specs=[pl.BlockSpec((BR, D), lambda i: (i, 0)),
                  pl.BlockSpec((D, D), lambda i: (0, 0))],
        out_specs=pl.BlockSpec((BR, D), lambda i: (i, 0)),
        out_shape=jax.ShapeDtypeStruct((NP, D), jnp.float32),
    )(x, w)


def _dinv_of(dpt_blk):
    deg = dpt_blk[:, 0:1] + dpt_blk[:, 1:2] + 1.0
    return lax.rsqrt(deg)


def _prep_body(dpt_ref, h_ref, hp_ref):
    hp_ref[...] = h_ref[...] * _dinv_of(dpt_ref[...])


def _prep(dpt, h):
    return pl.pallas_call(
        _prep_body,
        grid=(NB,),
        in_specs=[pl.BlockSpec((BR, 2), lambda i: (i, 0)),
                  pl.BlockSpec((BR, D), lambda i: (i, 0))],
        out_specs=pl.BlockSpec((BR, D), lambda i: (i, 0)),
        out_shape=jax.ShapeDtypeStruct((NP, D), jnp.float32),
    )(dpt, h)


def _pool_into(pooled_ref, bt, y, is_first):
    @pl.when(is_first)
    def _():
        pooled_ref[...] = jnp.full((G, D), -jnp.inf, dtype=jnp.float32)

    glo = jnp.min(bt)
    ghi = jnp.minimum(jnp.max(bt), G - 1)

    def pbody(g, carry):
        m = bt == g
        v = jnp.max(jnp.where(m, y, -jnp.inf), axis=0, keepdims=True)
        pooled_ref[pl.ds(g, 1), :] = jnp.maximum(pooled_ref[pl.ds(g, 1), :], v)
        return carry

    lax.fori_loop(glo, ghi + 1, pbody, 0)


def _layer1_body(dpt_ref, accp_ref, hp_ref, b_ref, w2_ref, bt_ref,
                 pooled_ref, hp2_ref):
    i = pl.program_id(0)
    dinv = _dinv_of(dpt_ref[...])
    y = jnp.maximum(
        dinv * (accp_ref[0] + accp_ref[1] + hp_ref[...]) + b_ref[...], 0.0)
    _pool_into(pooled_ref, bt_ref[...], y, i == 0)
    hp2_ref[...] = dinv * jnp.dot(y, w2_ref[...],
                                  preferred_element_type=jnp.float32)


def _layer1(dpt, accp, hp, b, w2, bt):
    return pl.pallas_call(
        _layer1_body,
        grid=(NB,),
        in_specs=[pl.BlockSpec((BR, 2), lambda i: (i, 0)),
                  pl.BlockSpec((NC, BR, D), lambda i: (0, i, 0)),
                  pl.BlockSpec((BR, D), lambda i: (i, 0)),
                  pl.BlockSpec((1, D), lambda i: (0, 0)),
                  pl.BlockSpec((D, D), lambda i: (0, 0)),
                  pl.BlockSpec((BR, 1), lambda i: (i, 0))],
        out_specs=[pl.BlockSpec((G, D), lambda i: (0, 0)),
                   pl.BlockSpec((BR, D), lambda i: (i, 0))],
        out_shape=[jax.ShapeDtypeStruct((G, D), jnp.float32),
                   jax.ShapeDtypeStruct((NP, D), jnp.float32)],
    )(dpt, accp, hp, b, w2, bt)


def _layer2_body(dpt_ref, accp_ref, hp_ref, b_ref, bt_ref, pooled_ref):
    i = pl.program_id(0)
    dinv = _dinv_of(dpt_ref[...])
    y = jnp.maximum(
        dinv * (accp_ref[0] + accp_ref[1] + hp_ref[...]) + b_ref[...], 0.0)
    _pool_into(pooled_ref, bt_ref[...], y, i == 0)


def _layer2(dpt, accp, hp, b, bt):
    return pl.pallas_call(
        _layer2_body,
        grid=(NB,),
        in_specs=[pl.BlockSpec((BR, 2), lambda i: (i, 0)),
                  pl.BlockSpec((NC, BR, D), lambda i: (0, i, 0)),
                  pl.BlockSpec((BR, D), lambda i: (i, 0)),
                  pl.BlockSpec((1, D), lambda i: (0, 0)),
                  pl.BlockSpec((BR, 1), lambda i: (i, 0))],
        out_specs=pl.BlockSpec((G, D), lambda i: (0, 0)),
        out_shape=jax.ShapeDtypeStruct((G, D), jnp.float32),
    )(dpt, accp, hp, b, bt)


def kernel(x, edge_index, edge_weight, batch, W1, b1, W2, b2):
    pad_e = EP - N_EDGES
    rowp = jnp.concatenate(
        [edge_index[0], jnp.zeros((pad_e,), jnp.int32)]).reshape(CP, K)
    colp = jnp.concatenate(
        [edge_index[1], jnp.zeros((pad_e,), jnp.int32)]).reshape(CP, K)
    ewp = jnp.concatenate(
        [edge_weight, jnp.zeros((pad_e,), jnp.float32)]).reshape(CP, K)
    xp = jnp.concatenate(
        [x, jnp.zeros((NP - N_NODES, D), x.dtype)], axis=0)
    btp = jnp.concatenate(
        [batch, jnp.full((NP - N_NODES,), G, jnp.int32)]).reshape(NP, 1)
    z1 = jnp.zeros((NP,), jnp.float32)
    z2 = jnp.zeros((NP, D), jnp.float32)
    b1r = b1.reshape(1, D)
    b2r = b2.reshape(1, D)

    dp = _deg_kernel(colp, ewp, z1)
    dpt = dp.T
    h1 = _mm(xp, W1)
    hp1 = _prep(dpt, h1)
    acc1 = _msg_kernel(rowp, colp, ewp, hp1, z2)
    pooled1, hp2 = _layer1(dpt, acc1, hp1, b1r, W2, btp)
    acc2 = _msg_kernel(rowp, colp, ewp, hp2, z2)
    pooled2 = _layer2(dpt, acc2, hp2, b2r, btp)
    return (pooled1, pooled2)
